# Initial kernel scaffold; baseline (speedup 1.0000x reference)
import jax
import jax.numpy as jnp
from jax.experimental import pallas as pl


def kernel(data_0, data_1, data_2, data_3, data_4, data_5, data_6, data_7, temporal_padding_mask):
    raise NotImplementedError("write your pallas kernel here")


# speedup vs baseline: 6.1822x; 6.1822x over previous
import functools

import jax
import jax.numpy as jnp
from jax.experimental import pallas as pl
from jax.experimental.pallas import tpu as pltpu

B, L, M, D = 4, 2048, 8, 768
NUM_REMAIN = 4
TL = 256


def _body(noise_ref, pm_ref, d0, d1, d2, d3, d4, d5, d6, d7,
          out_data_ref, out_rmask_ref, out_remain_ref, out_masked_ref,
          out_revert_ref):
    n = noise_ref[0]
    data = (d0, d1, d2, d3, d4, d5, d6, d7)

    ranks = []
    for m in range(M):
        nm = n[:, m:m + 1]
        acc = jnp.zeros((TL, 1), dtype=jnp.int32)
        for mp in range(M):
            if mp == m:
                continue
            nmp = n[:, mp:mp + 1]
            lt = nmp < nm
            if mp < m:
                lt = jnp.logical_or(lt, nmp == nm)
            acc = acc + lt.astype(jnp.int32)
        ranks.append(acc)

    out_revert_ref[0] = jnp.concatenate(ranks, axis=1)

    for r in range(NUM_REMAIN):
        rem = jnp.zeros((TL, 1), dtype=jnp.int32)
        msk = jnp.zeros((TL, 1), dtype=jnp.int32)
        for m in range(M):
            mi = jnp.int32(m)
            rem = rem + jnp.where(ranks[m] == r, mi, 0)
            msk = msk + jnp.where(ranks[m] == r + NUM_REMAIN, mi, 0)
        out_remain_ref[0, :, r:r + 1] = rem
        out_masked_ref[0, :, r:r + 1] = msk

    pmv = pm_ref[0]
    out_rmask_ref[0] = jnp.broadcast_to(pmv, (TL, NUM_REMAIN))

    for r in range(NUM_REMAIN):
        acc = jnp.zeros((TL, D), dtype=jnp.float32)
        for m in range(M):
            acc = jnp.where(ranks[m] == r, data[m][0], acc)
        out_data_ref[0, :, r * D:(r + 1) * D] = acc


@functools.partial(jax.jit, static_argnums=())
def _run(noise, pm, data):
    grid = (B, L // TL)
    data_spec = pl.BlockSpec((1, TL, D), lambda b, i: (b, i, 0))
    outs = pl.pallas_call(
        _body,
        grid=grid,
        in_specs=[
            pl.BlockSpec((1, TL, M), lambda b, i: (b, i, 0)),
            pl.BlockSpec((1, TL, 1), lambda b, i: (b, i, 0)),
        ] + [data_spec] * M,
        out_specs=[
            pl.BlockSpec((1, TL, NUM_REMAIN * D), lambda b, i: (b, i, 0)),
            pl.BlockSpec((1, TL, NUM_REMAIN), lambda b, i: (b, i, 0)),
            pl.BlockSpec((1, TL, NUM_REMAIN), lambda b, i: (b, i, 0)),
            pl.BlockSpec((1, TL, NUM_REMAIN), lambda b, i: (b, i, 0)),
            pl.BlockSpec((1, TL, M), lambda b, i: (b, i, 0)),
        ],
        out_shape=[
            jax.ShapeDtypeStruct((B, L, NUM_REMAIN * D), jnp.float32),
            jax.ShapeDtypeStruct((B, L, NUM_REMAIN), jnp.float32),
            jax.ShapeDtypeStruct((B, L, NUM_REMAIN), jnp.int32),
            jax.ShapeDtypeStruct((B, L, NUM_REMAIN), jnp.int32),
            jax.ShapeDtypeStruct((B, L, M), jnp.int32),
        ],
        compiler_params=pltpu.CompilerParams(
            dimension_semantics=("parallel", "parallel"),
        ),
    )(noise, pm, *data)
    return outs


def kernel(data_0, data_1, data_2, data_3, data_4, data_5, data_6, data_7,
           temporal_padding_mask):
    data = (data_0, data_1, data_2, data_3, data_4, data_5, data_6, data_7)
    noise = jax.random.uniform(jax.random.key(42), (B, L, M))
    pm = jnp.concatenate(
        [jnp.ones((B, 1, 1), temporal_padding_mask.dtype), temporal_padding_mask],
        axis=1)
    res = _run(noise, pm, data)
    remained_flat, remain_mask, remain_idx, masked_idx, revert_idx = res
    remained_data = remained_flat.reshape(B, L, NUM_REMAIN, D)
    return (remained_data, remain_mask, remain_idx, masked_idx, revert_idx, pm)
